# baseline (speedup 1.0000x reference)
import functools

import jax
import jax.numpy as jnp
from jax import lax
from jax.experimental import pallas as pl
from jax.experimental.pallas import tpu as pltpu
from jax.experimental.pallas import tpu_sc as plsc

B = 16384
D = 128
NC = 2
NS = 16
NW = NC * NS
BPW = B // NW
CHUNK = 128
NCHUNK = BPW // CHUNK
NGRP = CHUNK // 16


def _mf_body(users_hbm, items_hbm, ut_hbm, it_hbm, wb_hbm, out_hbm,
             uidx, iidx, u0, u1, i0, i1, wb_v, out_v, usem, isem):
    wid = lax.axis_index("s") * NC + lax.axis_index("c")
    base = wid * BPW

    pltpu.sync_copy(users_hbm.at[wid], uidx)
    pltpu.sync_copy(items_hbm.at[wid], iidx)
    pltpu.sync_copy(wb_hbm, wb_v)

    ubufs = (u0, u1)
    ibufs = (i0, i1)

    def start(c):
        ub = ubufs[c & 1]
        ib = ibufs[c & 1]
        hu = pltpu.async_copy(ut_hbm.at[uidx.at[c]], ub, usem)
        hi = pltpu.async_copy(it_hbm.at[iidx.at[c]], ib, isem)
        return hu, hi

    pending = start(0)

    riota = lax.iota(jnp.int32, 16)
    ridx = [riota + (g * 16) for g in range(NGRP)]
    bvec = wb_v[D]

    for c in range(NCHUNK):
        hu, hi = pending
        hu.wait()
        hi.wait()
        if c + 1 < NCHUNK:
            pending = start(c + 1)

        ub = ubufs[c & 1]
        ib = ibufs[c & 1]

        def dbody(d, accs, ub=ub, ib=ib):
            w_d = wb_v[d]
            col = jnp.full((16,), d, jnp.int32)
            out = []
            for g in range(NGRP):
                ug = plsc.load_gather(ub, [ridx[g], col])
                ig = plsc.load_gather(ib, [ridx[g], col])
                pb = plsc.bitcast(ug * ig, jnp.int32)
                lsb = lax.shift_right_logical(pb, 16) & 1
                pb = (pb + 0x7FFF + lsb) & jnp.int32(-65536)
                p = plsc.bitcast(pb, jnp.float32)
                out.append(accs[g] + p * w_d)
            return tuple(out)

        accs = lax.fori_loop(
            0, D, dbody,
            tuple(jnp.zeros((16,), jnp.float32) for _ in range(NGRP)))

        for g in range(NGRP):
            out_v[pl.ds(c * CHUNK + g * 16, 16)] = accs[g] + bvec

    pltpu.sync_copy(out_v, out_hbm.at[pl.ds(base, BPW)])


@jax.jit
def kernel(users, items, user_table, item_table, W, b):
    users_r = users.astype(jnp.int32).reshape(NW, NCHUNK, CHUNK)
    items_r = items.astype(jnp.int32).reshape(NW, NCHUNK, CHUNK)
    wi = lax.bitcast_convert_type(W.astype(jnp.float32), jnp.int32)
    wi = (wi + 0x7FFF + (lax.shift_right_logical(wi, 16) & 1)) & jnp.int32(-65536)
    w_rounded = lax.bitcast_convert_type(wi, jnp.float32)
    wb = jnp.concatenate(
        [jnp.broadcast_to(w_rounded, (D, 16)), jnp.broadcast_to(b, (1, 16))]
    ).astype(jnp.float32)

    mesh = plsc.VectorSubcoreMesh(core_axis_name="c", subcore_axis_name="s")
    out = pl.kernel(
        _mf_body,
        mesh=mesh,
        compiler_params=pltpu.CompilerParams(needs_layout_passes=False),
        out_type=jax.ShapeDtypeStruct((B,), jnp.float32),
        scratch_types=[
            pltpu.VMEM((NCHUNK, CHUNK), jnp.int32),
            pltpu.VMEM((NCHUNK, CHUNK), jnp.int32),
            pltpu.VMEM((CHUNK, D), jnp.float32),
            pltpu.VMEM((CHUNK, D), jnp.float32),
            pltpu.VMEM((CHUNK, D), jnp.float32),
            pltpu.VMEM((CHUNK, D), jnp.float32),
            pltpu.VMEM((D + 1, 16), jnp.float32),
            pltpu.VMEM((BPW,), jnp.float32),
            pltpu.SemaphoreType.DMA,
            pltpu.SemaphoreType.DMA,
        ],
    )(users_r, items_r, user_table, item_table, wb)
    return out.reshape(B, 1)


# speedup vs baseline: 2.0533x; 2.0533x over previous
import functools

import jax
import jax.numpy as jnp
from jax import lax
from jax.experimental import pallas as pl
from jax.experimental.pallas import tpu as pltpu
from jax.experimental.pallas import tpu_sc as plsc

B = 16384
D = 128
NC = 2
NS = 16
NW = NC * NS
BPW = B // NW
CHUNK = 128
NCHUNK = BPW // CHUNK
NGRP = CHUNK // 16


def _mf_body(users_hbm, items_hbm, ut_hbm, it_hbm, wb_hbm, out_hbm,
             uidx, iidx, u0, u1, i0, i1, wb_v, out_v, usem, isem):
    wid = lax.axis_index("s") * NC + lax.axis_index("c")
    base = wid * BPW

    pltpu.sync_copy(users_hbm.at[wid], uidx)
    pltpu.sync_copy(items_hbm.at[wid], iidx)
    pltpu.sync_copy(wb_hbm, wb_v)

    ubufs = (u0, u1)
    ibufs = (i0, i1)

    def start(c):
        ub = ubufs[c & 1]
        ib = ibufs[c & 1]
        hu = pltpu.async_copy(ut_hbm.at[uidx.at[c]], ub, usem)
        hi = pltpu.async_copy(it_hbm.at[iidx.at[c]], ib, isem)
        return hu, hi

    pending = start(0)

    riota = lax.iota(jnp.int32, 16)
    ridx = [riota + (g * 16) for g in range(NGRP)]
    bvec = wb_v[D]

    for c in range(NCHUNK):
        hu, hi = pending
        hu.wait()
        hi.wait()
        if c + 1 < NCHUNK:
            pending = start(c + 1)

        ub = ubufs[c & 1]
        ib = ibufs[c & 1]

        def dbody(d, accs, ub=ub, ib=ib):
            w_d = wb_v[d]
            col = (riota + d) & (D - 1)
            out = []
            for g in range(NGRP):
                ug = plsc.load_gather(ub, [ridx[g], col])
                ig = plsc.load_gather(ib, [ridx[g], col])
                pb = plsc.bitcast(ug * ig, jnp.int32)
                lsb = lax.shift_right_logical(pb, 16) & 1
                pb = (pb + 0x7FFF + lsb) & jnp.int32(-65536)
                p = plsc.bitcast(pb, jnp.float32)
                out.append(accs[g] + p * w_d)
            return tuple(out)

        accs = lax.fori_loop(
            0, D, dbody,
            tuple(jnp.zeros((16,), jnp.float32) for _ in range(NGRP)))

        for g in range(NGRP):
            out_v[pl.ds(c * CHUNK + g * 16, 16)] = accs[g] + bvec

    pltpu.sync_copy(out_v, out_hbm.at[pl.ds(base, BPW)])


@jax.jit
def kernel(users, items, user_table, item_table, W, b):
    users_r = users.astype(jnp.int32).reshape(NW, NCHUNK, CHUNK)
    items_r = items.astype(jnp.int32).reshape(NW, NCHUNK, CHUNK)
    wi = lax.bitcast_convert_type(W.astype(jnp.float32), jnp.int32)
    wi = (wi + 0x7FFF + (lax.shift_right_logical(wi, 16) & 1)) & jnp.int32(-65536)
    w_rounded = lax.bitcast_convert_type(wi, jnp.float32)[:, 0]
    skew = (jnp.arange(D)[:, None] + jnp.arange(16)[None, :]) % D
    wsk = w_rounded[skew]
    wb = jnp.concatenate([wsk, jnp.broadcast_to(b, (1, 16))]).astype(jnp.float32)

    mesh = plsc.VectorSubcoreMesh(core_axis_name="c", subcore_axis_name="s")
    out = pl.kernel(
        _mf_body,
        mesh=mesh,
        compiler_params=pltpu.CompilerParams(needs_layout_passes=False),
        out_type=jax.ShapeDtypeStruct((B,), jnp.float32),
        scratch_types=[
            pltpu.VMEM((NCHUNK, CHUNK), jnp.int32),
            pltpu.VMEM((NCHUNK, CHUNK), jnp.int32),
            pltpu.VMEM((CHUNK, D), jnp.float32),
            pltpu.VMEM((CHUNK, D), jnp.float32),
            pltpu.VMEM((CHUNK, D), jnp.float32),
            pltpu.VMEM((CHUNK, D), jnp.float32),
            pltpu.VMEM((D + 1, 16), jnp.float32),
            pltpu.VMEM((BPW,), jnp.float32),
            pltpu.SemaphoreType.DMA,
            pltpu.SemaphoreType.DMA,
        ],
    )(users_r, items_r, user_table, item_table, wb)
    return out.reshape(B, 1)


# speedup vs baseline: 2.0900x; 1.0179x over previous
import functools

import jax
import jax.numpy as jnp
from jax import lax
from jax.experimental import pallas as pl
from jax.experimental.pallas import tpu as pltpu
from jax.experimental.pallas import tpu_sc as plsc

B = 16384
D = 128
NC = 2
NS = 16
NW = NC * NS
BPW = B // NW
CHUNK = 128
NCHUNK = BPW // CHUNK
NGRP = CHUNK // 16


def _mf_body(users_hbm, items_hbm, ut_hbm, it_hbm, wb_hbm, out_hbm,
             uidx, iidx, u0, u1, i0, i1, wb_v, out_v, usem, isem):
    wid = lax.axis_index("s") * NC + lax.axis_index("c")
    base = wid * BPW

    pltpu.sync_copy(users_hbm.at[pl.ds(base, BPW)], uidx)
    pltpu.sync_copy(items_hbm.at[pl.ds(base, BPW)], iidx)
    pltpu.sync_copy(wb_hbm, wb_v)

    ubufs = (u0, u1)
    ibufs = (i0, i1)

    def start(c):
        ub = ubufs[c & 1]
        ib = ibufs[c & 1]
        hu = pltpu.async_copy(ut_hbm.at[uidx.at[pl.ds(c * CHUNK, CHUNK)]], ub, usem)
        hi = pltpu.async_copy(it_hbm.at[iidx.at[pl.ds(c * CHUNK, CHUNK)]], ib, isem)
        return hu, hi

    pending = start(0)

    riota = lax.iota(jnp.int32, 16)
    ridx = [riota + (g * 16) for g in range(NGRP)]
    bvec = wb_v[D]

    for c in range(NCHUNK):
        hu, hi = pending
        hu.wait()
        hi.wait()
        if c + 1 < NCHUNK:
            pending = start(c + 1)

        ub = ubufs[c & 1]
        ib = ibufs[c & 1]

        def dbody(d, accs, ub=ub, ib=ib):
            w_d = wb_v[d]
            col = (riota + d) & (D - 1)
            prods = []
            for g in range(NGRP):
                ug = plsc.load_gather(ub, [ridx[g], col])
                ig = plsc.load_gather(ib, [ridx[g], col])
                prods.append(ug * ig)
            out = []
            for g in range(0, NGRP, 2):
                packed = plsc.pack(prods[g], prods[g + 1],
                                   format=plsc.PackFormat.INTERLEAVED)
                p0, p1 = plsc.unpack(packed, format=plsc.PackFormat.INTERLEAVED,
                                     preferred_element_type=jnp.float32)
                out.append(accs[g] + p0 * w_d)
                out.append(accs[g + 1] + p1 * w_d)
            return tuple(out)

        accs = lax.fori_loop(
            0, D, dbody,
            tuple(jnp.zeros((16,), jnp.float32) for _ in range(NGRP)))

        for g in range(NGRP):
            out_v[pl.ds(c * CHUNK + g * 16, 16)] = accs[g] + bvec

    pltpu.sync_copy(out_v, out_hbm.at[pl.ds(base, BPW)])


@jax.jit
def kernel(users, items, user_table, item_table, W, b):
    users_r = users.astype(jnp.int32)
    items_r = items.astype(jnp.int32)
    wi = lax.bitcast_convert_type(W.astype(jnp.float32), jnp.int32)
    wi = (wi + 0x7FFF + (lax.shift_right_logical(wi, 16) & 1)) & jnp.int32(-65536)
    w_rounded = lax.bitcast_convert_type(wi, jnp.float32)[:, 0]
    skew = (jnp.arange(D)[:, None] + jnp.arange(16)[None, :]) % D
    wsk = w_rounded[skew]
    wb = jnp.concatenate([wsk, jnp.broadcast_to(b, (1, 16))]).astype(jnp.float32)

    mesh = plsc.VectorSubcoreMesh(core_axis_name="c", subcore_axis_name="s")
    out = pl.kernel(
        _mf_body,
        mesh=mesh,
        compiler_params=pltpu.CompilerParams(needs_layout_passes=False),
        out_type=jax.ShapeDtypeStruct((B,), jnp.float32),
        scratch_types=[
            pltpu.VMEM((BPW,), jnp.int32),
            pltpu.VMEM((BPW,), jnp.int32),
            pltpu.VMEM((CHUNK, D), jnp.float32),
            pltpu.VMEM((CHUNK, D), jnp.float32),
            pltpu.VMEM((CHUNK, D), jnp.float32),
            pltpu.VMEM((CHUNK, D), jnp.float32),
            pltpu.VMEM((D + 1, 16), jnp.float32),
            pltpu.VMEM((BPW,), jnp.float32),
            pltpu.SemaphoreType.DMA,
            pltpu.SemaphoreType.DMA,
        ],
    )(users_r, items_r, user_table, item_table, wb)
    return out.reshape(B, 1)
